# Initial kernel scaffold; baseline (speedup 1.0000x reference)
import jax
import jax.numpy as jnp
from jax.experimental import pallas as pl


def kernel(x, edge_index, W1, b1, W2, b2, W3, b3, W4, b4):
    raise NotImplementedError("write your pallas kernel here")


# speedup vs baseline: 7.5669x; 7.5669x over previous
import functools

import jax
import jax.numpy as jnp
from jax import lax
from jax.experimental import pallas as pl
from jax.experimental.pallas import tpu as pltpu
from jax.experimental.pallas import tpu_sc as plsc

N = 10000
NPAD = 10240
E = 320000
CH = 128
EPAD = 79 * 4096
EPT = EPAD // 16
EPT2 = EPAD // 32
ROWS_PT = NPAD // 16

_MESH = plsc.VectorSubcoreMesh(core_axis_name="c", subcore_axis_name="s")
_SC_PARAMS = pltpu.CompilerParams(needs_layout_passes=False)


def _deg_body(dst_hbm, hist_out, idx_v, hist_v):
    c = lax.axis_index("c")
    s = lax.axis_index("s")
    wid = c * 16 + s

    zero16 = jnp.zeros((16,), jnp.float32)

    def zf(i, _):
        hist_v[pl.ds(i * 16, 16)] = zero16
        return 0

    lax.fori_loop(0, NPAD // 16, zf, 0)

    ones16 = jnp.ones((16,), jnp.float32)
    epw = E // 32
    chunk = 2000

    def outer(k, _):
        pltpu.sync_copy(dst_hbm.at[pl.ds(wid * epw + k * chunk, chunk)], idx_v)

        def inner(j, _):
            iv = idx_v[pl.ds(j * 16, 16)]
            plsc.addupdate_scatter(hist_v, [iv], ones16)
            return 0

        lax.fori_loop(0, chunk // 16, inner, 0)
        return 0

    lax.fori_loop(0, epw // chunk, outer, 0)
    pltpu.sync_copy(hist_v, hist_out.at[wid])


@jax.jit
def _deg(dst_i32):
    return pl.kernel(
        _deg_body,
        out_type=jax.ShapeDtypeStruct((32, NPAD), jnp.float32),
        mesh=_MESH,
        scratch_types=[
            pltpu.VMEM((2000,), jnp.int32),
            pltpu.VMEM((NPAD,), jnp.float32),
        ],
        compiler_params=_SC_PARAMS,
    )(dst_i32)


def _zero_acc(zeros_hbm, acc, s):
    pltpu.sync_copy(zeros_hbm.at[pl.ds(s * ROWS_PT, ROWS_PT)],
                    acc.at[pl.ds(s * ROWS_PT, ROWS_PT)])
    plsc.subcore_barrier()


def _edge_loop(src_hbm, dst_hbm, g_hbm, acc, idx_s, idx_d, rows_v, sem,
               tile_base, n_iters):
    def body(i, _):
        ebase = tile_base + i * CH
        pltpu.sync_copy(src_hbm.at[pl.ds(ebase, CH)], idx_s)
        pltpu.sync_copy(dst_hbm.at[pl.ds(ebase, CH)], idx_d)
        pltpu.async_copy(g_hbm.at[idx_s], rows_v, sem).wait()
        pltpu.sync_copy(rows_v, acc.at[idx_d], add=True)
        return 0

    lax.fori_loop(0, n_iters, body, 0)


def _msg_col_body(src_hbm, dst_hbm, glo_hbm, ghi_hbm, zeros_hbm,
                  slo_hbm, shi_hbm, idx_s, idx_d, rows_v, acc, sem):
    c = lax.axis_index("c")
    s = lax.axis_index("s")
    _zero_acc(zeros_hbm, acc, s)

    @pl.when(c == 0)
    def _():
        _edge_loop(src_hbm, dst_hbm, glo_hbm, acc, idx_s, idx_d, rows_v, sem,
                   s * EPT, EPT // CH)

    @pl.when(c == 1)
    def _():
        _edge_loop(src_hbm, dst_hbm, ghi_hbm, acc, idx_s, idx_d, rows_v, sem,
                   s * EPT, EPT // CH)

    plsc.subcore_barrier()
    sl = pl.ds(s * ROWS_PT, ROWS_PT)

    @pl.when(c == 0)
    def _():
        pltpu.sync_copy(acc.at[sl], slo_hbm.at[sl])

    @pl.when(c == 1)
    def _():
        pltpu.sync_copy(acc.at[sl], shi_hbm.at[sl])


def _msg_edge_body(src_hbm, dst_hbm, g_hbm, zeros_hbm,
                   sa_hbm, sb_hbm, idx_s, idx_d, rows_v, acc, sem):
    c = lax.axis_index("c")
    s = lax.axis_index("s")
    _zero_acc(zeros_hbm, acc, s)
    _edge_loop(src_hbm, dst_hbm, g_hbm, acc, idx_s, idx_d, rows_v, sem,
               (c * 16 + s) * EPT2, EPT2 // CH)
    plsc.subcore_barrier()
    sl = pl.ds(s * ROWS_PT, ROWS_PT)

    @pl.when(c == 0)
    def _():
        pltpu.sync_copy(acc.at[sl], sa_hbm.at[sl])

    @pl.when(c == 1)
    def _():
        pltpu.sync_copy(acc.at[sl], sb_hbm.at[sl])


_OT128 = jax.ShapeDtypeStruct((NPAD, 128), jnp.float32)
_MSG_SCRATCH = [
    pltpu.VMEM((CH,), jnp.int32),
    pltpu.VMEM((CH,), jnp.int32),
    pltpu.VMEM((CH, 128), jnp.float32),
    pltpu.VMEM_SHARED((NPAD, 128), jnp.float32),
    pltpu.SemaphoreType.DMA,
]


@jax.jit
def _msg_col(src_p, dst_p, glo, ghi, zeros):
    return pl.kernel(
        _msg_col_body,
        out_type=(_OT128, _OT128),
        mesh=_MESH,
        scratch_types=_MSG_SCRATCH,
        compiler_params=_SC_PARAMS,
    )(src_p, dst_p, glo, ghi, zeros)


@jax.jit
def _msg_edge(src_p, dst_p, g, zeros):
    return pl.kernel(
        _msg_edge_body,
        out_type=(_OT128, _OT128),
        mesh=_MESH,
        scratch_types=_MSG_SCRATCH,
        compiler_params=_SC_PARAMS,
    )(src_p, dst_p, g, zeros)


_MB = 640
_GRID = NPAD // _MB


def _tc1_body(hist_ref, x_ref, w_ref, glo_ref, ghi_ref, dinv_ref):
    pid = pl.program_id(0)
    deg = jnp.sum(hist_ref[...], axis=0) + 1.0
    rid = lax.broadcasted_iota(jnp.int32, (_MB,), 0) + pid * _MB
    dinv = jnp.where(rid < N, lax.rsqrt(deg), 0.0)
    h = jnp.dot(x_ref[...], w_ref[...], preferred_element_type=jnp.float32)
    g = h * dinv[:, None]
    glo_ref[...] = g[:, :128]
    ghi_ref[...] = g[:, 128:]
    dinv_ref[...] = dinv[:, None]


@jax.jit
def _tc1(hist, x_pad, w1):
    return pl.pallas_call(
        _tc1_body,
        grid=(_GRID,),
        in_specs=[
            pl.BlockSpec((32, _MB), lambda i: (0, i)),
            pl.BlockSpec((_MB, 128), lambda i: (i, 0)),
            pl.BlockSpec((128, 256), lambda i: (0, 0)),
        ],
        out_specs=[
            pl.BlockSpec((_MB, 128), lambda i: (i, 0)),
            pl.BlockSpec((_MB, 128), lambda i: (i, 0)),
            pl.BlockSpec((_MB, 1), lambda i: (i, 0)),
        ],
        out_shape=[_OT128, _OT128,
                   jax.ShapeDtypeStruct((NPAD, 1), jnp.float32)],
    )(hist, x_pad, w1)


def _tc_cat_body(relu, slo_ref, shi_ref, glo_ref, ghi_ref, dinv_ref, b_ref,
                 w_ref, o_ref):
    sg = jnp.concatenate([slo_ref[...] + glo_ref[...],
                          shi_ref[...] + ghi_ref[...]], axis=1)
    dinv = dinv_ref[...]
    a = dinv * sg + b_ref[...][None, :]
    if relu:
        a = jnp.maximum(a, 0.0)
    h = jnp.dot(a, w_ref[...], preferred_element_type=jnp.float32)
    g = h * dinv
    if g.shape[1] < 128:
        g = jnp.concatenate(
            [g, jnp.zeros((_MB, 128 - g.shape[1]), jnp.float32)], axis=1)
    o_ref[...] = g


@functools.partial(jax.jit, static_argnums=(7,))
def _tc_cat(slo, shi, glo, ghi, dinv, b, w, relu):
    din, dout = w.shape
    return pl.pallas_call(
        functools.partial(_tc_cat_body, relu),
        grid=(_GRID,),
        in_specs=[
            pl.BlockSpec((_MB, 128), lambda i: (i, 0)),
            pl.BlockSpec((_MB, 128), lambda i: (i, 0)),
            pl.BlockSpec((_MB, 128), lambda i: (i, 0)),
            pl.BlockSpec((_MB, 128), lambda i: (i, 0)),
            pl.BlockSpec((_MB, 1), lambda i: (i, 0)),
            pl.BlockSpec((din,), lambda i: (0,)),
            pl.BlockSpec((din, dout), lambda i: (0, 0)),
        ],
        out_specs=pl.BlockSpec((_MB, 128), lambda i: (i, 0)),
        out_shape=_OT128,
    )(slo, shi, glo, ghi, dinv, b, w)


def _tc_sum_body(relu, sa_ref, sb_ref, g_ref, dinv_ref, b_ref, w_ref,
                 olo_ref, ohi_ref):
    sg = sa_ref[...] + sb_ref[...] + g_ref[...]
    dinv = dinv_ref[...]
    a = dinv * sg + b_ref[...][None, :]
    if relu:
        a = jnp.maximum(a, 0.0)
    h = jnp.dot(a, w_ref[...], preferred_element_type=jnp.float32)
    g = h * dinv
    olo_ref[...] = g[:, :128]
    ohi_ref[...] = g[:, 128:]


@functools.partial(jax.jit, static_argnums=(6,))
def _tc_sum(sa, sb, g, dinv, b, w, relu):
    return pl.pallas_call(
        functools.partial(_tc_sum_body, relu),
        grid=(_GRID,),
        in_specs=[
            pl.BlockSpec((_MB, 128), lambda i: (i, 0)),
            pl.BlockSpec((_MB, 128), lambda i: (i, 0)),
            pl.BlockSpec((_MB, 128), lambda i: (i, 0)),
            pl.BlockSpec((_MB, 1), lambda i: (i, 0)),
            pl.BlockSpec((128,), lambda i: (0,)),
            pl.BlockSpec((128, 256), lambda i: (0, 0)),
        ],
        out_specs=[
            pl.BlockSpec((_MB, 128), lambda i: (i, 0)),
            pl.BlockSpec((_MB, 128), lambda i: (i, 0)),
        ],
        out_shape=[_OT128, _OT128],
    )(sa, sb, g, dinv, b, w)


def _tc_fin_body(sa_ref, sb_ref, g_ref, dinv_ref, b_ref, o_ref):
    o_ref[...] = (dinv_ref[...] * (sa_ref[...] + sb_ref[...] + g_ref[...])
                  + b_ref[...][None, :])


@jax.jit
def _tc_fin(sa, sb, g, dinv, b):
    return pl.pallas_call(
        _tc_fin_body,
        grid=(_GRID,),
        in_specs=[
            pl.BlockSpec((_MB, 128), lambda i: (i, 0)),
            pl.BlockSpec((_MB, 128), lambda i: (i, 0)),
            pl.BlockSpec((_MB, 128), lambda i: (i, 0)),
            pl.BlockSpec((_MB, 1), lambda i: (i, 0)),
            pl.BlockSpec((128,), lambda i: (0,)),
        ],
        out_specs=pl.BlockSpec((_MB, 128), lambda i: (i, 0)),
        out_shape=_OT128,
    )(sa, sb, g, dinv, b)


def kernel(x, edge_index, W1, b1, W2, b2, W3, b3, W4, b4):
    src = edge_index[0].astype(jnp.int32)
    dst = edge_index[1].astype(jnp.int32)
    padi = jnp.full((EPAD - E,), N, jnp.int32)
    src_p = jnp.concatenate([src, padi])
    dst_p = jnp.concatenate([dst, padi])
    x_pad = jnp.zeros((NPAD, 128), jnp.float32).at[:N].set(x)
    z128 = jnp.zeros((NPAD, 128), jnp.float32)
    b2p = jnp.concatenate([b2, jnp.zeros((64,), jnp.float32)])
    W3p = jnp.concatenate([W3, jnp.zeros((64, 256), jnp.float32)], axis=0)

    hist = _deg(dst)
    g1lo, g1hi, dinv = _tc1(hist, x_pad, W1)
    s1lo, s1hi = _msg_col(src_p, dst_p, g1lo, g1hi, z128)
    g2 = _tc_cat(s1lo, s1hi, g1lo, g1hi, dinv, b1, W2, True)
    s2a, s2b = _msg_edge(src_p, dst_p, g2, z128)
    g3lo, g3hi = _tc_sum(s2a, s2b, g2, dinv, b2p, W3p, False)
    s3lo, s3hi = _msg_col(src_p, dst_p, g3lo, g3hi, z128)
    g4 = _tc_cat(s3lo, s3hi, g3lo, g3hi, dinv, b3, W4, True)
    s4a, s4b = _msg_edge(src_p, dst_p, g4, z128)
    xr = _tc_fin(s4a, s4b, g4, dinv, b4)
    return xr[:N]


# speedup vs baseline: 7.6810x; 1.0151x over previous
import functools

import jax
import jax.numpy as jnp
from jax import lax
from jax.experimental import pallas as pl
from jax.experimental.pallas import tpu as pltpu
from jax.experimental.pallas import tpu_sc as plsc

N = 10000
NPAD = 10240
E = 320000
CH = 128
EPAD = 80 * 4096
NCH_COL = EPAD // (16 * CH)
NCH_EDGE = EPAD // (32 * CH)
NIB = 8
ROWS_PT = NPAD // 16

_MESH = plsc.VectorSubcoreMesh(core_axis_name="c", subcore_axis_name="s")
_SC_PARAMS = pltpu.CompilerParams(needs_layout_passes=False)


def _deg_body(dst_hbm, hist_out, idx_v, hist_v):
    c = lax.axis_index("c")
    s = lax.axis_index("s")
    wid = c * 16 + s

    zero16 = jnp.zeros((16,), jnp.float32)

    def zf(i, _):
        hist_v[pl.ds(i * 16, 16)] = zero16
        return 0

    lax.fori_loop(0, NPAD // 16, zf, 0)

    ones16 = jnp.ones((16,), jnp.float32)
    epw = E // 32
    chunk = 2000

    def outer(k, _):
        pltpu.sync_copy(dst_hbm.at[pl.ds(wid * epw + k * chunk, chunk)], idx_v)

        def inner(j, _):
            iv = idx_v[pl.ds(j * 16, 16)]
            plsc.addupdate_scatter(hist_v, [iv], ones16)
            return 0

        lax.fori_loop(0, chunk // 16, inner, 0)
        return 0

    lax.fori_loop(0, epw // chunk, outer, 0)
    pltpu.sync_copy(hist_v, hist_out.at[wid])


@jax.jit
def _deg(dst_i32):
    return pl.kernel(
        _deg_body,
        out_type=jax.ShapeDtypeStruct((32, NPAD), jnp.float32),
        mesh=_MESH,
        scratch_types=[
            pltpu.VMEM((2000,), jnp.int32),
            pltpu.VMEM((NPAD,), jnp.float32),
        ],
        compiler_params=_SC_PARAMS,
    )(dst_i32)


def _zero_acc(zeros_hbm, acc, s):
    pltpu.sync_copy(zeros_hbm.at[pl.ds(s * ROWS_PT, ROWS_PT)],
                    acc.at[pl.ds(s * ROWS_PT, ROWS_PT)])
    plsc.subcore_barrier()


def _edge_loop(src2_hbm, dst2_hbm, g_hbm, acc, idx_s, idx_d, rows_v, sems,
               chunk_base, n_chunks):
    si = sems[0:2]
    sg = sems[2:4]
    ss = sems[4:6]
    nblk = n_chunks // NIB

    pltpu.async_copy(src2_hbm.at[pl.ds(chunk_base, NIB)], idx_s.at[0], si[0])
    pltpu.async_copy(dst2_hbm.at[pl.ds(chunk_base, NIB)], idx_d.at[0], si[0])

    def do_block(p, cur):
        nb = p * 2 + cur
        blk = chunk_base + nb * NIB
        pltpu.make_async_copy(src2_hbm.at[pl.ds(blk, NIB)], idx_s.at[cur],
                              si[cur]).wait()
        pltpu.make_async_copy(dst2_hbm.at[pl.ds(blk, NIB)], idx_d.at[cur],
                              si[cur]).wait()

        @pl.when(nb + 1 < nblk)
        def _():
            pltpu.async_copy(src2_hbm.at[pl.ds(blk + NIB, NIB)],
                             idx_s.at[1 - cur], si[1 - cur])
            pltpu.async_copy(dst2_hbm.at[pl.ds(blk + NIB, NIB)],
                             idx_d.at[1 - cur], si[1 - cur])

        pltpu.async_copy(g_hbm.at[idx_s.at[cur].at[0]], rows_v.at[0], sg[0])
        pltpu.async_copy(g_hbm.at[idx_s.at[cur].at[1]], rows_v.at[1], sg[1])
        for j in range(NIB):
            b = j & 1
            pltpu.make_async_copy(g_hbm.at[idx_s.at[cur].at[j]],
                                  rows_v.at[b], sg[b]).wait()
            pltpu.async_copy(rows_v.at[b], acc.at[idx_d.at[cur].at[j]],
                             ss[b], add=True)
            if j + 2 < NIB:
                pltpu.make_async_copy(rows_v.at[b],
                                      acc.at[idx_d.at[cur].at[j]],
                                      ss[b]).wait()
                pltpu.async_copy(g_hbm.at[idx_s.at[cur].at[j + 2]],
                                 rows_v.at[b], sg[b])
        for j in (NIB - 2, NIB - 1):
            b = j & 1
            pltpu.make_async_copy(rows_v.at[b],
                                  acc.at[idx_d.at[cur].at[j]], ss[b]).wait()

    def pair(p, _):
        do_block(p, 0)
        do_block(p, 1)
        return 0

    lax.fori_loop(0, nblk // 2, pair, 0)


def _msg_col_body(src2_hbm, dst2_hbm, glo_hbm, ghi_hbm, zeros_hbm,
                  slo_hbm, shi_hbm, idx_s, idx_d, rows_v, acc, *sems):
    c = lax.axis_index("c")
    s = lax.axis_index("s")
    _zero_acc(zeros_hbm, acc, s)

    @pl.when(c == 0)
    def _():
        _edge_loop(src2_hbm, dst2_hbm, glo_hbm, acc, idx_s, idx_d, rows_v,
                   sems, s * NCH_COL, NCH_COL)

    @pl.when(c == 1)
    def _():
        _edge_loop(src2_hbm, dst2_hbm, ghi_hbm, acc, idx_s, idx_d, rows_v,
                   sems, s * NCH_COL, NCH_COL)

    plsc.subcore_barrier()
    sl = pl.ds(s * ROWS_PT, ROWS_PT)

    @pl.when(c == 0)
    def _():
        pltpu.sync_copy(acc.at[sl], slo_hbm.at[sl])

    @pl.when(c == 1)
    def _():
        pltpu.sync_copy(acc.at[sl], shi_hbm.at[sl])


def _msg_edge_body(src2_hbm, dst2_hbm, g_hbm, zeros_hbm,
                   sa_hbm, sb_hbm, idx_s, idx_d, rows_v, acc, *sems):
    c = lax.axis_index("c")
    s = lax.axis_index("s")
    _zero_acc(zeros_hbm, acc, s)
    _edge_loop(src2_hbm, dst2_hbm, g_hbm, acc, idx_s, idx_d, rows_v,
               sems, (c * 16 + s) * NCH_EDGE, NCH_EDGE)
    plsc.subcore_barrier()
    sl = pl.ds(s * ROWS_PT, ROWS_PT)

    @pl.when(c == 0)
    def _():
        pltpu.sync_copy(acc.at[sl], sa_hbm.at[sl])

    @pl.when(c == 1)
    def _():
        pltpu.sync_copy(acc.at[sl], sb_hbm.at[sl])


_OT128 = jax.ShapeDtypeStruct((NPAD, 128), jnp.float32)
_MSG_SCRATCH = [
    pltpu.VMEM((2, NIB, CH), jnp.int32),
    pltpu.VMEM((2, NIB, CH), jnp.int32),
    pltpu.VMEM((2, CH, 128), jnp.float32),
    pltpu.VMEM_SHARED((NPAD, 128), jnp.float32),
] + [pltpu.SemaphoreType.DMA] * 6


@jax.jit
def _msg_col(src_p, dst_p, glo, ghi, zeros):
    return pl.kernel(
        _msg_col_body,
        out_type=(_OT128, _OT128),
        mesh=_MESH,
        scratch_types=_MSG_SCRATCH,
        compiler_params=_SC_PARAMS,
    )(src_p, dst_p, glo, ghi, zeros)


@jax.jit
def _msg_edge(src_p, dst_p, g, zeros):
    return pl.kernel(
        _msg_edge_body,
        out_type=(_OT128, _OT128),
        mesh=_MESH,
        scratch_types=_MSG_SCRATCH,
        compiler_params=_SC_PARAMS,
    )(src_p, dst_p, g, zeros)


_MB = 640
_GRID = NPAD // _MB


def _tc1_body(hist_ref, x_ref, w_ref, glo_ref, ghi_ref, dinv_ref):
    pid = pl.program_id(0)
    deg = jnp.sum(hist_ref[...], axis=0) + 1.0
    rid = lax.broadcasted_iota(jnp.int32, (_MB,), 0) + pid * _MB
    dinv = jnp.where(rid < N, lax.rsqrt(deg), 0.0)
    h = jnp.dot(x_ref[...], w_ref[...], preferred_element_type=jnp.float32)
    g = h * dinv[:, None]
    glo_ref[...] = g[:, :128]
    ghi_ref[...] = g[:, 128:]
    dinv_ref[...] = dinv[:, None]


@jax.jit
def _tc1(hist, x_pad, w1):
    return pl.pallas_call(
        _tc1_body,
        grid=(_GRID,),
        in_specs=[
            pl.BlockSpec((32, _MB), lambda i: (0, i)),
            pl.BlockSpec((_MB, 128), lambda i: (i, 0)),
            pl.BlockSpec((128, 256), lambda i: (0, 0)),
        ],
        out_specs=[
            pl.BlockSpec((_MB, 128), lambda i: (i, 0)),
            pl.BlockSpec((_MB, 128), lambda i: (i, 0)),
            pl.BlockSpec((_MB, 1), lambda i: (i, 0)),
        ],
        out_shape=[_OT128, _OT128,
                   jax.ShapeDtypeStruct((NPAD, 1), jnp.float32)],
    )(hist, x_pad, w1)


def _tc_cat_body(relu, slo_ref, shi_ref, glo_ref, ghi_ref, dinv_ref, b_ref,
                 w_ref, o_ref):
    sg = jnp.concatenate([slo_ref[...] + glo_ref[...],
                          shi_ref[...] + ghi_ref[...]], axis=1)
    dinv = dinv_ref[...]
    a = dinv * sg + b_ref[...][None, :]
    if relu:
        a = jnp.maximum(a, 0.0)
    h = jnp.dot(a, w_ref[...], preferred_element_type=jnp.float32)
    g = h * dinv
    if g.shape[1] < 128:
        g = jnp.concatenate(
            [g, jnp.zeros((_MB, 128 - g.shape[1]), jnp.float32)], axis=1)
    o_ref[...] = g


@functools.partial(jax.jit, static_argnums=(7,))
def _tc_cat(slo, shi, glo, ghi, dinv, b, w, relu):
    din, dout = w.shape
    return pl.pallas_call(
        functools.partial(_tc_cat_body, relu),
        grid=(_GRID,),
        in_specs=[
            pl.BlockSpec((_MB, 128), lambda i: (i, 0)),
            pl.BlockSpec((_MB, 128), lambda i: (i, 0)),
            pl.BlockSpec((_MB, 128), lambda i: (i, 0)),
            pl.BlockSpec((_MB, 128), lambda i: (i, 0)),
            pl.BlockSpec((_MB, 1), lambda i: (i, 0)),
            pl.BlockSpec((din,), lambda i: (0,)),
            pl.BlockSpec((din, dout), lambda i: (0, 0)),
        ],
        out_specs=pl.BlockSpec((_MB, 128), lambda i: (i, 0)),
        out_shape=_OT128,
    )(slo, shi, glo, ghi, dinv, b, w)


def _tc_sum_body(relu, sa_ref, sb_ref, g_ref, dinv_ref, b_ref, w_ref,
                 olo_ref, ohi_ref):
    sg = sa_ref[...] + sb_ref[...] + g_ref[...]
    dinv = dinv_ref[...]
    a = dinv * sg + b_ref[...][None, :]
    if relu:
        a = jnp.maximum(a, 0.0)
    h = jnp.dot(a, w_ref[...], preferred_element_type=jnp.float32)
    g = h * dinv
    olo_ref[...] = g[:, :128]
    ohi_ref[...] = g[:, 128:]


@functools.partial(jax.jit, static_argnums=(6,))
def _tc_sum(sa, sb, g, dinv, b, w, relu):
    return pl.pallas_call(
        functools.partial(_tc_sum_body, relu),
        grid=(_GRID,),
        in_specs=[
            pl.BlockSpec((_MB, 128), lambda i: (i, 0)),
            pl.BlockSpec((_MB, 128), lambda i: (i, 0)),
            pl.BlockSpec((_MB, 128), lambda i: (i, 0)),
            pl.BlockSpec((_MB, 1), lambda i: (i, 0)),
            pl.BlockSpec((128,), lambda i: (0,)),
            pl.BlockSpec((128, 256), lambda i: (0, 0)),
        ],
        out_specs=[
            pl.BlockSpec((_MB, 128), lambda i: (i, 0)),
            pl.BlockSpec((_MB, 128), lambda i: (i, 0)),
        ],
        out_shape=[_OT128, _OT128],
    )(sa, sb, g, dinv, b, w)


def _tc_fin_body(sa_ref, sb_ref, g_ref, dinv_ref, b_ref, o_ref):
    o_ref[...] = (dinv_ref[...] * (sa_ref[...] + sb_ref[...] + g_ref[...])
                  + b_ref[...][None, :])


@jax.jit
def _tc_fin(sa, sb, g, dinv, b):
    return pl.pallas_call(
        _tc_fin_body,
        grid=(_GRID,),
        in_specs=[
            pl.BlockSpec((_MB, 128), lambda i: (i, 0)),
            pl.BlockSpec((_MB, 128), lambda i: (i, 0)),
            pl.BlockSpec((_MB, 128), lambda i: (i, 0)),
            pl.BlockSpec((_MB, 1), lambda i: (i, 0)),
            pl.BlockSpec((128,), lambda i: (0,)),
        ],
        out_specs=pl.BlockSpec((_MB, 128), lambda i: (i, 0)),
        out_shape=_OT128,
    )(sa, sb, g, dinv, b)


def kernel(x, edge_index, W1, b1, W2, b2, W3, b3, W4, b4):
    src = edge_index[0].astype(jnp.int32)
    dst = edge_index[1].astype(jnp.int32)
    padi = jnp.full((EPAD - E,), N, jnp.int32)
    src_p = jnp.concatenate([src, padi]).reshape(EPAD // CH, CH)
    dst_p = jnp.concatenate([dst, padi]).reshape(EPAD // CH, CH)
    x_pad = jnp.zeros((NPAD, 128), jnp.float32).at[:N].set(x)
    z128 = jnp.zeros((NPAD, 128), jnp.float32)
    b2p = jnp.concatenate([b2, jnp.zeros((64,), jnp.float32)])
    W3p = jnp.concatenate([W3, jnp.zeros((64, 256), jnp.float32)], axis=0)

    hist = _deg(dst)
    g1lo, g1hi, dinv = _tc1(hist, x_pad, W1)
    s1lo, s1hi = _msg_col(src_p, dst_p, g1lo, g1hi, z128)
    g2 = _tc_cat(s1lo, s1hi, g1lo, g1hi, dinv, b1, W2, True)
    s2a, s2b = _msg_edge(src_p, dst_p, g2, z128)
    g3lo, g3hi = _tc_sum(s2a, s2b, g2, dinv, b2p, W3p, False)
    s3lo, s3hi = _msg_col(src_p, dst_p, g3lo, g3hi, z128)
    g4 = _tc_cat(s3lo, s3hi, g3lo, g3hi, dinv, b3, W4, True)
    s4a, s4b = _msg_edge(src_p, dst_p, g4, z128)
    xr = _tc_fin(s4a, s4b, g4, dinv, b4)
    return xr[:N]


# speedup vs baseline: 7.7785x; 1.0127x over previous
import functools

import jax
import jax.numpy as jnp
from jax import lax
from jax.experimental import pallas as pl
from jax.experimental.pallas import tpu as pltpu
from jax.experimental.pallas import tpu_sc as plsc

N = 10000
NPAD = 10240
E = 320000
CH = 128
EPAD = 80 * 4096
NCH_COL = EPAD // (16 * CH)
NCH_EDGE = EPAD // (32 * CH)
NIB = 8
ROWS_PT = NPAD // 16

_MESH = plsc.VectorSubcoreMesh(core_axis_name="c", subcore_axis_name="s")
_SC_PARAMS = pltpu.CompilerParams(needs_layout_passes=False)


def _deg_body(dst_hbm, hist_out, idx_v, hist_v):
    c = lax.axis_index("c")
    s = lax.axis_index("s")
    wid = c * 16 + s

    zero16 = jnp.zeros((16,), jnp.float32)

    def zf(i, _):
        hist_v[pl.ds(i * 16, 16)] = zero16
        return 0

    lax.fori_loop(0, NPAD // 16, zf, 0)

    ones16 = jnp.ones((16,), jnp.float32)
    epw = E // 32
    chunk = 2000

    def outer(k, _):
        pltpu.sync_copy(dst_hbm.at[pl.ds(wid * epw + k * chunk, chunk)], idx_v)

        def inner(j, _):
            iv = idx_v[pl.ds(j * 16, 16)]
            plsc.addupdate_scatter(hist_v, [iv], ones16)
            return 0

        lax.fori_loop(0, chunk // 16, inner, 0)
        return 0

    lax.fori_loop(0, epw // chunk, outer, 0)
    pltpu.sync_copy(hist_v, hist_out.at[wid])


@jax.jit
def _deg(dst_i32):
    return pl.kernel(
        _deg_body,
        out_type=jax.ShapeDtypeStruct((32, NPAD), jnp.float32),
        mesh=_MESH,
        scratch_types=[
            pltpu.VMEM((2000,), jnp.int32),
            pltpu.VMEM((NPAD,), jnp.float32),
        ],
        compiler_params=_SC_PARAMS,
    )(dst_i32)


def _zero_acc(zeros_hbm, acc, s):
    pltpu.sync_copy(zeros_hbm.at[pl.ds(s * ROWS_PT, ROWS_PT)],
                    acc.at[pl.ds(s * ROWS_PT, ROWS_PT)])
    plsc.subcore_barrier()


def _edge_loop(src2_hbm, dst2_hbm, g_hbm, acc, idx_s, idx_d, rows_v, sems,
               chunk_base, n_chunks, nib):
    si = sems[0:2]
    sg = sems[2:4]
    ss = sems[4:6]
    nblk = n_chunks // nib

    pltpu.async_copy(src2_hbm.at[pl.ds(chunk_base, nib)],
                     idx_s.at[0].at[pl.ds(0, nib)], si[0])
    pltpu.async_copy(dst2_hbm.at[pl.ds(chunk_base, nib)],
                     idx_d.at[0].at[pl.ds(0, nib)], si[0])

    def do_block(p, cur):
        nb = p * 2 + cur
        blk = chunk_base + nb * nib
        pltpu.make_async_copy(src2_hbm.at[pl.ds(blk, nib)],
                              idx_s.at[cur].at[pl.ds(0, nib)], si[cur]).wait()
        pltpu.make_async_copy(dst2_hbm.at[pl.ds(blk, nib)],
                              idx_d.at[cur].at[pl.ds(0, nib)], si[cur]).wait()

        @pl.when(nb + 1 < nblk)
        def _():
            pltpu.async_copy(src2_hbm.at[pl.ds(blk + nib, nib)],
                             idx_s.at[1 - cur].at[pl.ds(0, nib)],
                             si[1 - cur])
            pltpu.async_copy(dst2_hbm.at[pl.ds(blk + nib, nib)],
                             idx_d.at[1 - cur].at[pl.ds(0, nib)],
                             si[1 - cur])

        pltpu.async_copy(g_hbm.at[idx_s.at[cur].at[0]], rows_v.at[0], sg[0])
        pltpu.async_copy(g_hbm.at[idx_s.at[cur].at[1]], rows_v.at[1], sg[1])
        for j in range(nib):
            b = j & 1
            pltpu.make_async_copy(g_hbm.at[idx_s.at[cur].at[j]],
                                  rows_v.at[b], sg[b]).wait()
            pltpu.async_copy(rows_v.at[b], acc.at[idx_d.at[cur].at[j]],
                             ss[b], add=True)
            if j + 2 < nib:
                pltpu.make_async_copy(rows_v.at[b],
                                      acc.at[idx_d.at[cur].at[j]],
                                      ss[b]).wait()
                pltpu.async_copy(g_hbm.at[idx_s.at[cur].at[j + 2]],
                                 rows_v.at[b], sg[b])
        for j in (nib - 2, nib - 1):
            b = j & 1
            pltpu.make_async_copy(rows_v.at[b],
                                  acc.at[idx_d.at[cur].at[j]], ss[b]).wait()

    def pair(p, _):
        do_block(p, 0)
        do_block(p, 1)
        return 0

    lax.fori_loop(0, nblk // 2, pair, 0)


def _msg_col_body(src2_hbm, dst2_hbm, glo_hbm, ghi_hbm, zeros_hbm,
                  slo_hbm, shi_hbm, idx_s, idx_d, rows_v, acc, *sems):
    c = lax.axis_index("c")
    s = lax.axis_index("s")
    _zero_acc(zeros_hbm, acc, s)

    @pl.when(c == 0)
    def _():
        _edge_loop(src2_hbm, dst2_hbm, glo_hbm, acc, idx_s, idx_d, rows_v,
                   sems, s * NCH_COL, NCH_COL, 16)

    @pl.when(c == 1)
    def _():
        _edge_loop(src2_hbm, dst2_hbm, ghi_hbm, acc, idx_s, idx_d, rows_v,
                   sems, s * NCH_COL, NCH_COL, 16)

    plsc.subcore_barrier()
    sl = pl.ds(s * ROWS_PT, ROWS_PT)

    @pl.when(c == 0)
    def _():
        pltpu.sync_copy(acc.at[sl], slo_hbm.at[sl])

    @pl.when(c == 1)
    def _():
        pltpu.sync_copy(acc.at[sl], shi_hbm.at[sl])


def _msg_edge_body(src2_hbm, dst2_hbm, g_hbm, zeros_hbm,
                   sa_hbm, sb_hbm, idx_s, idx_d, rows_v, acc, *sems):
    c = lax.axis_index("c")
    s = lax.axis_index("s")
    _zero_acc(zeros_hbm, acc, s)
    _edge_loop(src2_hbm, dst2_hbm, g_hbm, acc, idx_s, idx_d, rows_v,
               sems, (c * 16 + s) * NCH_EDGE, NCH_EDGE, 8)
    plsc.subcore_barrier()
    sl = pl.ds(s * ROWS_PT, ROWS_PT)

    @pl.when(c == 0)
    def _():
        pltpu.sync_copy(acc.at[sl], sa_hbm.at[sl])

    @pl.when(c == 1)
    def _():
        pltpu.sync_copy(acc.at[sl], sb_hbm.at[sl])


_OT128 = jax.ShapeDtypeStruct((NPAD, 128), jnp.float32)
_MSG_SCRATCH = [
    pltpu.VMEM((2, 16, CH), jnp.int32),
    pltpu.VMEM((2, 16, CH), jnp.int32),
    pltpu.VMEM((2, CH, 128), jnp.float32),
    pltpu.VMEM_SHARED((NPAD, 128), jnp.float32),
] + [pltpu.SemaphoreType.DMA] * 6


@jax.jit
def _msg_col(src_p, dst_p, glo, ghi, zeros):
    return pl.kernel(
        _msg_col_body,
        out_type=(_OT128, _OT128),
        mesh=_MESH,
        scratch_types=_MSG_SCRATCH,
        compiler_params=_SC_PARAMS,
    )(src_p, dst_p, glo, ghi, zeros)


@jax.jit
def _msg_edge(src_p, dst_p, g, zeros):
    return pl.kernel(
        _msg_edge_body,
        out_type=(_OT128, _OT128),
        mesh=_MESH,
        scratch_types=_MSG_SCRATCH,
        compiler_params=_SC_PARAMS,
    )(src_p, dst_p, g, zeros)


_MB = 640
_GRID = NPAD // _MB


def _tc1_body(hist_ref, x_ref, w_ref, glo_ref, ghi_ref, dinv_ref):
    pid = pl.program_id(0)
    deg = jnp.sum(hist_ref[...], axis=0) + 1.0
    rid = lax.broadcasted_iota(jnp.int32, (_MB,), 0) + pid * _MB
    dinv = jnp.where(rid < N, lax.rsqrt(deg), 0.0)
    h = jnp.dot(x_ref[...], w_ref[...], preferred_element_type=jnp.float32)
    g = h * dinv[:, None]
    glo_ref[...] = g[:, :128]
    ghi_ref[...] = g[:, 128:]
    dinv_ref[...] = dinv[:, None]


@jax.jit
def _tc1(hist, x_pad, w1):
    return pl.pallas_call(
        _tc1_body,
        grid=(_GRID,),
        in_specs=[
            pl.BlockSpec((32, _MB), lambda i: (0, i)),
            pl.BlockSpec((_MB, 128), lambda i: (i, 0)),
            pl.BlockSpec((128, 256), lambda i: (0, 0)),
        ],
        out_specs=[
            pl.BlockSpec((_MB, 128), lambda i: (i, 0)),
            pl.BlockSpec((_MB, 128), lambda i: (i, 0)),
            pl.BlockSpec((_MB, 1), lambda i: (i, 0)),
        ],
        out_shape=[_OT128, _OT128,
                   jax.ShapeDtypeStruct((NPAD, 1), jnp.float32)],
    )(hist, x_pad, w1)


def _tc_cat_body(relu, slo_ref, shi_ref, glo_ref, ghi_ref, dinv_ref, b_ref,
                 w_ref, o_ref):
    sg = jnp.concatenate([slo_ref[...] + glo_ref[...],
                          shi_ref[...] + ghi_ref[...]], axis=1)
    dinv = dinv_ref[...]
    a = dinv * sg + b_ref[...][None, :]
    if relu:
        a = jnp.maximum(a, 0.0)
    h = jnp.dot(a, w_ref[...], preferred_element_type=jnp.float32)
    g = h * dinv
    if g.shape[1] < 128:
        g = jnp.concatenate(
            [g, jnp.zeros((_MB, 128 - g.shape[1]), jnp.float32)], axis=1)
    o_ref[...] = g


@functools.partial(jax.jit, static_argnums=(7,))
def _tc_cat(slo, shi, glo, ghi, dinv, b, w, relu):
    din, dout = w.shape
    return pl.pallas_call(
        functools.partial(_tc_cat_body, relu),
        grid=(_GRID,),
        in_specs=[
            pl.BlockSpec((_MB, 128), lambda i: (i, 0)),
            pl.BlockSpec((_MB, 128), lambda i: (i, 0)),
            pl.BlockSpec((_MB, 128), lambda i: (i, 0)),
            pl.BlockSpec((_MB, 128), lambda i: (i, 0)),
            pl.BlockSpec((_MB, 1), lambda i: (i, 0)),
            pl.BlockSpec((din,), lambda i: (0,)),
            pl.BlockSpec((din, dout), lambda i: (0, 0)),
        ],
        out_specs=pl.BlockSpec((_MB, 128), lambda i: (i, 0)),
        out_shape=_OT128,
    )(slo, shi, glo, ghi, dinv, b, w)


def _tc_sum_body(relu, sa_ref, sb_ref, g_ref, dinv_ref, b_ref, w_ref,
                 olo_ref, ohi_ref):
    sg = sa_ref[...] + sb_ref[...] + g_ref[...]
    dinv = dinv_ref[...]
    a = dinv * sg + b_ref[...][None, :]
    if relu:
        a = jnp.maximum(a, 0.0)
    h = jnp.dot(a, w_ref[...], preferred_element_type=jnp.float32)
    g = h * dinv
    olo_ref[...] = g[:, :128]
    ohi_ref[...] = g[:, 128:]


@functools.partial(jax.jit, static_argnums=(6,))
def _tc_sum(sa, sb, g, dinv, b, w, relu):
    return pl.pallas_call(
        functools.partial(_tc_sum_body, relu),
        grid=(_GRID,),
        in_specs=[
            pl.BlockSpec((_MB, 128), lambda i: (i, 0)),
            pl.BlockSpec((_MB, 128), lambda i: (i, 0)),
            pl.BlockSpec((_MB, 128), lambda i: (i, 0)),
            pl.BlockSpec((_MB, 1), lambda i: (i, 0)),
            pl.BlockSpec((128,), lambda i: (0,)),
            pl.BlockSpec((128, 256), lambda i: (0, 0)),
        ],
        out_specs=[
            pl.BlockSpec((_MB, 128), lambda i: (i, 0)),
            pl.BlockSpec((_MB, 128), lambda i: (i, 0)),
        ],
        out_shape=[_OT128, _OT128],
    )(sa, sb, g, dinv, b, w)


def _tc_fin_body(sa_ref, sb_ref, g_ref, dinv_ref, b_ref, o_ref):
    o_ref[...] = (dinv_ref[...] * (sa_ref[...] + sb_ref[...] + g_ref[...])
                  + b_ref[...][None, :])


@jax.jit
def _tc_fin(sa, sb, g, dinv, b):
    return pl.pallas_call(
        _tc_fin_body,
        grid=(_GRID,),
        in_specs=[
            pl.BlockSpec((_MB, 128), lambda i: (i, 0)),
            pl.BlockSpec((_MB, 128), lambda i: (i, 0)),
            pl.BlockSpec((_MB, 128), lambda i: (i, 0)),
            pl.BlockSpec((_MB, 1), lambda i: (i, 0)),
            pl.BlockSpec((128,), lambda i: (0,)),
        ],
        out_specs=pl.BlockSpec((_MB, 128), lambda i: (i, 0)),
        out_shape=_OT128,
    )(sa, sb, g, dinv, b)


def kernel(x, edge_index, W1, b1, W2, b2, W3, b3, W4, b4):
    src = edge_index[0].astype(jnp.int32)
    dst = edge_index[1].astype(jnp.int32)
    padi = jnp.full((EPAD - E,), N, jnp.int32)
    src_p = jnp.concatenate([src, padi]).reshape(EPAD // CH, CH)
    dst_p = jnp.concatenate([dst, padi]).reshape(EPAD // CH, CH)
    x_pad = jnp.zeros((NPAD, 128), jnp.float32).at[:N].set(x)
    z128 = jnp.zeros((NPAD, 128), jnp.float32)
    b2p = jnp.concatenate([b2, jnp.zeros((64,), jnp.float32)])
    W3p = jnp.concatenate([W3, jnp.zeros((64, 256), jnp.float32)], axis=0)

    hist = _deg(dst)
    g1lo, g1hi, dinv = _tc1(hist, x_pad, W1)
    s1lo, s1hi = _msg_col(src_p, dst_p, g1lo, g1hi, z128)
    g2 = _tc_cat(s1lo, s1hi, g1lo, g1hi, dinv, b1, W2, True)
    s2a, s2b = _msg_edge(src_p, dst_p, g2, z128)
    g3lo, g3hi = _tc_sum(s2a, s2b, g2, dinv, b2p, W3p, False)
    s3lo, s3hi = _msg_col(src_p, dst_p, g3lo, g3hi, z128)
    g4 = _tc_cat(s3lo, s3hi, g3lo, g3hi, dinv, b3, W4, True)
    s4a, s4b = _msg_edge(src_p, dst_p, g4, z128)
    xr = _tc_fin(s4a, s4b, g4, dinv, b4)
    return xr[:N]
